# baseline (speedup 1.0000x reference)
import jax
import jax.numpy as jnp
from jax import lax
from jax.experimental import pallas as pl
from jax.experimental.pallas import tpu as pltpu
from jax.experimental.pallas import tpu_sc as plsc

_B = 16384
_DIMS = (8, 8, 8, 2, 4, 3, 64, 32, 16)
_VOCABS = (1000000, 100000, 1000, 2, 10, 5, 64, 32, 16)
_NSC = 3
_NC = 2
_NS = 16
_NW = _NC * _NS
_BPW = _B // _NW
_CHUNK = 128
_L = 16

_NTILES = tuple(-(-(_VOCABS[t] + 1) // 128) for t in range(_NSC))
_KT = 16


def _detile(ut, utail, pt, ptail, mt):
    mesh = plsc.VectorSubcoreMesh(core_axis_name="c", subcore_axis_name="s")
    out_type = tuple(jax.ShapeDtypeStruct((n * 8, 128), jnp.float32)
                     for n in _NTILES)

    def body(u_hbm, ut_hbm, p_hbm, pt_hbm, m_hbm, uo, po, mo,
             t0_v, t1_v, s0_v, s1_v, semi0, semi1, semo0, semo1):
        wid = lax.axis_index("s") * _NC + lax.axis_index("c")
        lanes = lax.iota(jnp.int32, _L)
        dvec = lanes % 8
        kbase = lanes // 8
        tiles = (t0_v, t1_v)
        sts = (s0_v, s1_v)
        semis = (semi0, semi1)
        semos = (semo0, semo1)

        colvecs = [lanes * 8 + d for d in range(8)]

        def transpose_tiles(tiles_v, st_v, n):
            def tbody(k, _):
                for q in range(8):
                    row = jnp.full((_L,), k * 8 + q, jnp.int32)
                    for d in range(8):
                        vals = tiles_v[d, pl.ds(k * 128 + 16 * q, _L)]
                        plsc.store_scatter(st_v, [row, colvecs[d]], vals)
                return 0
            lax.fori_loop(0, n, tbody, 0)

        def chunk(src, dst, src_t0, dst_t0, n):
            pltpu.sync_copy(
                src.at[:, pl.ds(pl.multiple_of(src_t0 * 128, 128), n * 128)],
                t0_v.at[:, pl.ds(0, n * 128)])
            transpose_tiles(t0_v, s0_v, n)
            pltpu.sync_copy(
                s0_v.at[pl.ds(0, n * 8), :],
                dst.at[pl.ds(pl.multiple_of(dst_t0 * 8, 8), n * 8), :])

        def chunk_loop(src, dst, nchunks, spw):
            def mk_in(s, b):
                c = pl.multiple_of((s * _NW + wid) * _KT * 128, 128)
                return pltpu.make_async_copy(
                    src.at[:, pl.ds(c, _KT * 128)], tiles[b], semis[b])

            def mk_out(s, b):
                c = pl.multiple_of((s * _NW + wid) * _KT * 8, 8)
                return pltpu.make_async_copy(
                    sts[b], dst.at[pl.ds(c, _KT * 8), :], semos[b])

            for b in range(2):
                @pl.when(b * _NW + wid < nchunks)
                def _():
                    mk_in(b, b).start()

            def sbody(s2, _):
                for b in range(2):
                    s = 2 * s2 + b
                    c = s * _NW + wid
                    @pl.when(c < nchunks)
                    def _():
                        mk_in(s, b).wait()
                        @pl.when(s >= 2)
                        def _():
                            mk_out(s - 2, b).wait()
                        transpose_tiles(tiles[b], sts[b], _KT)
                        mk_out(s, b).start()
                        @pl.when((s + 2) * _NW + wid < nchunks)
                        def _():
                            mk_in(s + 2, b).start()
                return 0
            lax.fori_loop(0, spw // 2, sbody, 0)
            for s in range(max(0, spw - 3), spw):
                @pl.when((s * _NW + wid < nchunks)
                         & ((s + 2) * _NW + wid >= nchunks))
                def _():
                    mk_out(s, s % 2).wait()

        chunk_loop(u_hbm, uo, 488, 16)
        @pl.when(wid == 0)
        def _():
            chunk(ut_hbm, uo, 0, 488 * _KT, 5)
        chunk_loop(p_hbm, po, 48, 2)
        @pl.when(wid == 1)
        def _():
            chunk(pt_hbm, po, 0, 48 * _KT, 14)
        @pl.when(wid == 2)
        def _():
            chunk(m_hbm, mo, 0, 0, 8)

    fn = pl.kernel(
        body, out_type=out_type, mesh=mesh,
        scratch_types=[pltpu.VMEM((8, _KT * 128), jnp.float32),
                       pltpu.VMEM((8, _KT * 128), jnp.float32),
                       pltpu.VMEM((_KT * 8, 128), jnp.float32),
                       pltpu.VMEM((_KT * 8, 128), jnp.float32),
                       pltpu.SemaphoreType.DMA,
                       pltpu.SemaphoreType.DMA,
                       pltpu.SemaphoreType.DMA,
                       pltpu.SemaphoreType.DMA],
        compiler_params=pltpu.CompilerParams(
            use_tc_tiling_on_sc=True, needs_layout_passes=False))
    return fn(ut, utail, pt, ptail, mt)


def _sc_gather(xt, *tables):
    mesh = plsc.VectorSubcoreMesh(core_axis_name="c", subcore_axis_name="s")
    out_type = jax.ShapeDtypeStruct((_B, 8 * _NSC), jnp.float32)
    scratch = (
        [pltpu.VMEM((_BPW,), jnp.float32),
         pltpu.VMEM((_NSC, _BPW // _CHUNK, _CHUNK), jnp.int32)]
        + [pltpu.VMEM((_BPW, 8), jnp.float32) for _ in range(_NSC)]
        + [pltpu.SemaphoreType.DMA]
    )

    def body(xt_hbm, *refs):
        tabs = refs[:_NSC]
        out = refs[_NSC]
        xcol = refs[_NSC + 1]
        idx_v = refs[_NSC + 2]
        rows = refs[_NSC + 3:_NSC + 3 + _NSC]
        sem = refs[_NSC + 3 + _NSC]

        wid = lax.axis_index("s") * _NC + lax.axis_index("c")
        base = wid * _BPW
        for t in range(_NSC):
            pltpu.sync_copy(xt_hbm.at[t, pl.ds(base, _BPW)], xcol)
            for c in range(_BPW // _L):
                vals = xcol[pl.ds(c * _L, _L)].astype(jnp.int32)
                j, o = divmod(c * _L, _CHUNK)
                idx_v[t, j, pl.ds(o, _L)] = vals
        copies = []
        for t in range(_NSC):
            for j in range(_BPW // _CHUNK):
                copies.append(pltpu.async_copy(
                    tabs[t].at[idx_v.at[t, j]],
                    rows[t].at[pl.ds(j * _CHUNK, _CHUNK), :], sem))
        for cp in copies:
            cp.wait()
        for t in range(_NSC):
            pltpu.sync_copy(
                rows[t], out.at[pl.ds(base, _BPW), pl.ds(t * 8, 8)])

    fn = pl.kernel(
        body, out_type=out_type, mesh=mesh, scratch_types=scratch,
        compiler_params=pltpu.CompilerParams(
            use_tc_tiling_on_sc=False, needs_layout_passes=False))
    return fn(xt, *tables)


_OFFS2 = []
_o2 = 0
for _d2 in _DIMS:
    _OFFS2.append(_o2)
    _o2 += _d2
_TIME_ROW = _o2


def _tc_pre(xt, small_t, W1t, b1):
    blk = 2048
    grid = (_B // blk,)

    def body(*refs):
        x_ref = refs[0]
        tt = refs[1:7]
        w1t, b1r = refs[7:9]
        out = refs[9]
        xb = x_ref[...]
        w1v = w1t[...]
        h = b1r[...] + w1v[:, _TIME_ROW:_TIME_ROW + 1] * xb[9:10, :]
        for k in range(6):
            t = _NSC + k
            v = _VOCABS[t] + 1
            projT = jnp.dot(w1v[:, _OFFS2[t]:_OFFS2[t] + _DIMS[t]], tt[k][...],
                            preferred_element_type=jnp.float32)
            ids = lax.broadcasted_iota(jnp.int32, (v, blk), 0)
            onehotT = (ids == xb[t:t + 1, :].astype(jnp.int32)).astype(jnp.float32)
            h = h + jnp.dot(projT, onehotT, preferred_element_type=jnp.float32)
        out[...] = h

    in_specs = (
        [pl.BlockSpec((10, blk), lambda i: (0, i))]
        + [pl.BlockSpec(t.shape, lambda i: (0, 0)) for t in small_t]
        + [pl.BlockSpec((16, 146), lambda i: (0, 0)),
           pl.BlockSpec((16, 1), lambda i: (0, 0))]
    )
    return pl.pallas_call(
        body,
        grid=grid,
        in_specs=in_specs,
        out_specs=pl.BlockSpec((16, blk), lambda i: (0, i)),
        out_shape=jax.ShapeDtypeStruct((16, _B), jnp.float32),
    )(xt, *small_t, W1t, b1)


def _tc_post(hpre, gt, W1t, W2t, b2, W3t, b3):
    blk = 2048
    grid = (_B // blk,)

    def body(h_ref, g_ref, w1t, w2t, b2r, w3t, b3r, out):
        gb = g_ref[...]
        w1v = w1t[...]
        h = h_ref[...] + lax.dot_general(
            w1v[:, 0:24], gb, (((1,), (1,)), ((), ())),
            preferred_element_type=jnp.float32)
        h = jnp.maximum(h, 0.0)
        h = jnp.maximum(jnp.dot(w2t[...], h, preferred_element_type=jnp.float32)
                        + b2r[...], 0.0)
        o1 = jnp.dot(w3t[...], h, preferred_element_type=jnp.float32) + b3r[...]
        out[...] = jnp.broadcast_to(jax.nn.sigmoid(o1), (8, blk))

    in_specs = [pl.BlockSpec((16, blk), lambda i: (0, i)),
                pl.BlockSpec((blk, 8 * _NSC), lambda i: (i, 0)),
                pl.BlockSpec((16, 146), lambda i: (0, 0)),
                pl.BlockSpec((4, 16), lambda i: (0, 0)),
                pl.BlockSpec((4, 1), lambda i: (0, 0)),
                pl.BlockSpec((1, 4), lambda i: (0, 0)),
                pl.BlockSpec((1, 1), lambda i: (0, 0))]
    return pl.pallas_call(
        body,
        grid=grid,
        in_specs=in_specs,
        out_specs=pl.BlockSpec((8, blk), lambda i: (0, i)),
        out_shape=jax.ShapeDtypeStruct((8, _B), jnp.float32),
    )(hpre, gt, W1t, W2t, b2, W3t, b3)


def kernel(x, user_emb, product_emb, model_emb, gender_emb, age_emb,
           residence_emb, color_emb, size_emb, material_emb,
           W1, b1, W2, b2, W3, b3):
    xt = x.T
    utail = jnp.pad(user_emb[488 * _KT * 128:], ((0, 5 * 128 - 577), (0, 0)))
    ptail = jnp.pad(product_emb[48 * _KT * 128:], ((0, 14 * 128 - 1697), (0, 0)))
    mpad = jnp.pad(model_emb, ((0, 8 * 128 - 1001), (0, 0)))
    uo, po, mo = _detile(user_emb.T, utail.T, product_emb.T, ptail.T, mpad.T)
    big = tuple(o.reshape(n * 8 * 16, 8) for o, n in zip((uo, po, mo), _NTILES))
    gt = _sc_gather(xt, *big)
    small_t = (gender_emb.T, age_emb.T, residence_emb.T, color_emb.T,
               size_emb.T, material_emb.T)
    hpre = _tc_pre(xt, small_t, W1.T, b1.reshape(16, 1))
    out = _tc_post(hpre, gt, W1.T, W2.T, b2.reshape(4, 1), W3.T,
                   b3.reshape(1, 1))
    return out[0]


# speedup vs baseline: 1.0121x; 1.0121x over previous
import jax
import jax.numpy as jnp
from jax import lax
from jax.experimental import pallas as pl
from jax.experimental.pallas import tpu as pltpu
from jax.experimental.pallas import tpu_sc as plsc

_B = 16384
_DIMS = (8, 8, 8, 2, 4, 3, 64, 32, 16)
_VOCABS = (1000000, 100000, 1000, 2, 10, 5, 64, 32, 16)
_NSC = 3
_NC = 2
_NS = 16
_NW = _NC * _NS
_BPW = _B // _NW
_CHUNK = 128
_L = 16

_NTILES = tuple(-(-(_VOCABS[t] + 1) // 128) for t in range(_NSC))
_KT = 16


def _detile(ut, utail, pt, ptail, mt):
    mesh = plsc.VectorSubcoreMesh(core_axis_name="c", subcore_axis_name="s")
    out_type = tuple(jax.ShapeDtypeStruct((n * 8, 128), jnp.float32)
                     for n in _NTILES)

    def body(u_hbm, ut_hbm, p_hbm, pt_hbm, m_hbm, uo, po, mo,
             t0_v, t1_v, s0_v, s1_v, semi0, semi1, semo0, semo1):
        wid = lax.axis_index("s") * _NC + lax.axis_index("c")
        lanes = lax.iota(jnp.int32, _L)
        dvec = lanes % 8
        kbase = lanes // 8
        tiles = (t0_v, t1_v)
        sts = (s0_v, s1_v)
        semis = (semi0, semi1)
        semos = (semo0, semo1)

        colvecs = [lanes * 8 + d for d in range(8)]

        def transpose_tiles(tiles_v, st_v, n):
            def tbody(k, _):
                for q in range(8):
                    row = jnp.full((_L,), k * 8 + q, jnp.int32)
                    for d in range(8):
                        vals = tiles_v[d, pl.ds(k * 128 + 16 * q, _L)]
                        plsc.store_scatter(st_v, [row, colvecs[d]], vals)
                return 0
            lax.fori_loop(0, n, tbody, 0)

        def chunk(src, dst, src_t0, dst_t0, n):
            pltpu.sync_copy(
                src.at[:, pl.ds(pl.multiple_of(src_t0 * 128, 128), n * 128)],
                t0_v.at[:, pl.ds(0, n * 128)])
            transpose_tiles(t0_v, s0_v, n)
            pltpu.sync_copy(
                s0_v.at[pl.ds(0, n * 8), :],
                dst.at[pl.ds(pl.multiple_of(dst_t0 * 8, 8), n * 8), :])

        def chunk_loop(src, dst, nchunks, spw):
            def mk_in(s, b):
                c = pl.multiple_of((s * _NW + wid) * _KT * 128, 128)
                return pltpu.make_async_copy(
                    src.at[:, pl.ds(c, _KT * 128)], tiles[b], semis[b])

            def mk_out(s, b):
                c = pl.multiple_of((s * _NW + wid) * _KT * 8, 8)
                return pltpu.make_async_copy(
                    sts[b], dst.at[pl.ds(c, _KT * 8), :], semos[b])

            for b in range(2):
                @pl.when(b * _NW + wid < nchunks)
                def _():
                    mk_in(b, b).start()

            def sbody(s2, _):
                for b in range(2):
                    s = 2 * s2 + b
                    c = s * _NW + wid
                    @pl.when(c < nchunks)
                    def _():
                        mk_in(s, b).wait()
                        @pl.when(s >= 2)
                        def _():
                            mk_out(s - 2, b).wait()
                        transpose_tiles(tiles[b], sts[b], _KT)
                        mk_out(s, b).start()
                        @pl.when((s + 2) * _NW + wid < nchunks)
                        def _():
                            mk_in(s + 2, b).start()
                return 0
            lax.fori_loop(0, spw // 2, sbody, 0)
            for s in range(max(0, spw - 3), spw):
                @pl.when((s * _NW + wid < nchunks)
                         & ((s + 2) * _NW + wid >= nchunks))
                def _():
                    mk_out(s, s % 2).wait()

        chunk_loop(u_hbm, uo, 488, 16)
        @pl.when(wid == 0)
        def _():
            chunk(ut_hbm, uo, 0, 488 * _KT, 5)
        chunk_loop(p_hbm, po, 48, 2)
        @pl.when(wid == 1)
        def _():
            chunk(pt_hbm, po, 0, 48 * _KT, 14)
        @pl.when(wid == 2)
        def _():
            chunk(m_hbm, mo, 0, 0, 8)

    fn = pl.kernel(
        body, out_type=out_type, mesh=mesh,
        scratch_types=[pltpu.VMEM((8, _KT * 128), jnp.float32),
                       pltpu.VMEM((8, _KT * 128), jnp.float32),
                       pltpu.VMEM((_KT * 8, 128), jnp.float32),
                       pltpu.VMEM((_KT * 8, 128), jnp.float32),
                       pltpu.SemaphoreType.DMA,
                       pltpu.SemaphoreType.DMA,
                       pltpu.SemaphoreType.DMA,
                       pltpu.SemaphoreType.DMA],
        compiler_params=pltpu.CompilerParams(
            use_tc_tiling_on_sc=True, needs_layout_passes=False))
    return fn(ut, utail, pt, ptail, mt)


def _sc_gather(xt, *tables):
    mesh = plsc.VectorSubcoreMesh(core_axis_name="c", subcore_axis_name="s")
    out_type = jax.ShapeDtypeStruct((8 * _NSC, _B), jnp.float32)
    scratch = (
        [pltpu.VMEM((_BPW,), jnp.float32),
         pltpu.VMEM((_NSC, _BPW // _CHUNK, _CHUNK), jnp.int32)]
        + [pltpu.VMEM((_BPW, 8), jnp.float32) for _ in range(_NSC)]
        + [pltpu.VMEM((8 * _NSC, _BPW), jnp.float32),
           pltpu.SemaphoreType.DMA]
    )

    def body(xt_hbm, *refs):
        tabs = refs[:_NSC]
        out = refs[_NSC]
        xcol = refs[_NSC + 1]
        idx_v = refs[_NSC + 2]
        rows = refs[_NSC + 3:_NSC + 3 + _NSC]
        st = refs[_NSC + 3 + _NSC]
        sem = refs[_NSC + 4 + _NSC]

        wid = lax.axis_index("s") * _NC + lax.axis_index("c")
        base = wid * _BPW
        lanes = lax.iota(jnp.int32, _L)
        for t in range(_NSC):
            pltpu.sync_copy(xt_hbm.at[t, pl.ds(base, _BPW)], xcol)
            for c in range(_BPW // _L):
                vals = xcol[pl.ds(c * _L, _L)].astype(jnp.int32)
                j, o = divmod(c * _L, _CHUNK)
                idx_v[t, j, pl.ds(o, _L)] = vals
        copies = []
        for t in range(_NSC):
            for j in range(_BPW // _CHUNK):
                copies.append(pltpu.async_copy(
                    tabs[t].at[idx_v.at[t, j]],
                    rows[t].at[pl.ds(j * _CHUNK, _CHUNK), :], sem))
        for cp in copies:
            cp.wait()
        for t in range(_NSC):
            for d in range(8):
                dv = jnp.full((_L,), d, jnp.int32)
                for g in range(_BPW // _L):
                    vals = plsc.load_gather(rows[t], [g * _L + lanes, dv])
                    st[t * 8 + d, pl.ds(g * _L, _L)] = vals
        pltpu.sync_copy(st, out.at[:, pl.ds(base, _BPW)])

    fn = pl.kernel(
        body, out_type=out_type, mesh=mesh, scratch_types=scratch,
        compiler_params=pltpu.CompilerParams(
            use_tc_tiling_on_sc=False, needs_layout_passes=False))
    return fn(xt, *tables)


_OFFS2 = []
_o2 = 0
for _d2 in _DIMS:
    _OFFS2.append(_o2)
    _o2 += _d2
_TIME_ROW = _o2


def _tc_pre(xt, small_t, W1t, b1):
    blk = 2048
    grid = (_B // blk,)

    def body(*refs):
        x_ref = refs[0]
        tt = refs[1:7]
        w1t, b1r = refs[7:9]
        out = refs[9]
        xb = x_ref[...]
        w1v = w1t[...]
        h = b1r[...] + w1v[:, _TIME_ROW:_TIME_ROW + 1] * xb[9:10, :]
        for k in range(6):
            t = _NSC + k
            v = _VOCABS[t] + 1
            projT = jnp.dot(w1v[:, _OFFS2[t]:_OFFS2[t] + _DIMS[t]], tt[k][...],
                            preferred_element_type=jnp.float32)
            ids = lax.broadcasted_iota(jnp.int32, (v, blk), 0)
            onehotT = (ids == xb[t:t + 1, :].astype(jnp.int32)).astype(jnp.float32)
            h = h + jnp.dot(projT, onehotT, preferred_element_type=jnp.float32)
        out[...] = h

    in_specs = (
        [pl.BlockSpec((10, blk), lambda i: (0, i))]
        + [pl.BlockSpec(t.shape, lambda i: (0, 0)) for t in small_t]
        + [pl.BlockSpec((16, 146), lambda i: (0, 0)),
           pl.BlockSpec((16, 1), lambda i: (0, 0))]
    )
    return pl.pallas_call(
        body,
        grid=grid,
        in_specs=in_specs,
        out_specs=pl.BlockSpec((16, blk), lambda i: (0, i)),
        out_shape=jax.ShapeDtypeStruct((16, _B), jnp.float32),
    )(xt, *small_t, W1t, b1)


def _tc_post(hpre, gt, W1t, W2t, b2, W3t, b3):
    blk = 2048
    grid = (_B // blk,)

    def body(h_ref, g_ref, w1t, w2t, b2r, w3t, b3r, out):
        gb = g_ref[...]
        w1v = w1t[...]
        h = h_ref[...] + jnp.dot(w1v[:, 0:24], gb,
                                 preferred_element_type=jnp.float32)
        h = jnp.maximum(h, 0.0)
        h = jnp.maximum(jnp.dot(w2t[...], h, preferred_element_type=jnp.float32)
                        + b2r[...], 0.0)
        o1 = jnp.dot(w3t[...], h, preferred_element_type=jnp.float32) + b3r[...]
        out[...] = jnp.broadcast_to(jax.nn.sigmoid(o1), (8, blk))

    in_specs = [pl.BlockSpec((16, blk), lambda i: (0, i)),
                pl.BlockSpec((8 * _NSC, blk), lambda i: (0, i)),
                pl.BlockSpec((16, 146), lambda i: (0, 0)),
                pl.BlockSpec((4, 16), lambda i: (0, 0)),
                pl.BlockSpec((4, 1), lambda i: (0, 0)),
                pl.BlockSpec((1, 4), lambda i: (0, 0)),
                pl.BlockSpec((1, 1), lambda i: (0, 0))]
    return pl.pallas_call(
        body,
        grid=grid,
        in_specs=in_specs,
        out_specs=pl.BlockSpec((8, blk), lambda i: (0, i)),
        out_shape=jax.ShapeDtypeStruct((8, _B), jnp.float32),
    )(hpre, gt, W1t, W2t, b2, W3t, b3)


def kernel(x, user_emb, product_emb, model_emb, gender_emb, age_emb,
           residence_emb, color_emb, size_emb, material_emb,
           W1, b1, W2, b2, W3, b3):
    xt = x.T
    utail = jnp.pad(user_emb[488 * _KT * 128:], ((0, 5 * 128 - 577), (0, 0)))
    ptail = jnp.pad(product_emb[48 * _KT * 128:], ((0, 14 * 128 - 1697), (0, 0)))
    mpad = jnp.pad(model_emb, ((0, 8 * 128 - 1001), (0, 0)))
    uo, po, mo = _detile(user_emb.T, utail.T, product_emb.T, ptail.T, mpad.T)
    big = tuple(o.reshape(n * 8 * 16, 8) for o, n in zip((uo, po, mo), _NTILES))
    gt = _sc_gather(xt, *big)
    small_t = (gender_emb.T, age_emb.T, residence_emb.T, color_emb.T,
               size_emb.T, material_emb.T)
    hpre = _tc_pre(xt, small_t, W1.T, b1.reshape(16, 1))
    out = _tc_post(hpre, gt, W1.T, W2.T, b2.reshape(4, 1), W3.T,
                   b3.reshape(1, 1))
    return out[0]
